# baseline (speedup 1.0000x reference)
import functools

import jax
import jax.numpy as jnp
from jax import lax
from jax.experimental import pallas as pl
from jax.experimental.pallas import tpu as pltpu
from jax.experimental.pallas import tpu_sc as plsc

N = 10000
E = 320000
D = 128
NGRAPH = 64

NC = 2
NS = 16
NW = NC * NS

NROW = 10112
RPT = NROW // NS

CW = 125
EPW = E // NW
EWP = 18000
NCHUNK = EWP // CW
PC = 24
NPIECE = NCHUNK // PC
DEAD = 10016


def _sc_segment_body(h_hbm, zeros_hbm, src_hbm, dst_hbm, out_hbm,
                     src_v, dst_v, rows_v, acc_sh, gsem):
    c = lax.axis_index("c")
    s = lax.axis_index("s")

    pltpu.sync_copy(zeros_hbm.at[pl.ds(s * RPT, RPT)],
                    acc_sh.at[pl.ds(s * RPT, RPT)])
    plsc.subcore_barrier()

    wid = c * NS + s

    def piece(p, carry):
        pltpu.sync_copy(src_hbm.at[wid, pl.ds(p * PC, PC)], src_v)
        pltpu.sync_copy(dst_hbm.at[wid, pl.ds(p * PC, PC)], dst_v)

        def body(j, carry2):
            pltpu.async_copy(h_hbm.at[src_v.at[j]], rows_v, gsem).wait()
            pltpu.sync_copy(rows_v, acc_sh.at[dst_v.at[j]], add=True)
            return carry2

        return lax.fori_loop(0, PC, body, carry)

    lax.fori_loop(0, NPIECE, piece, 0)

    plsc.subcore_barrier()
    pltpu.sync_copy(acc_sh.at[pl.ds(s * RPT, RPT)],
                    out_hbm.at[c, pl.ds(s * RPT, RPT)])


_sc_segment = functools.partial(
    pl.kernel,
    out_type=jax.ShapeDtypeStruct((NC, NROW, D), jnp.float32),
    mesh=plsc.VectorSubcoreMesh(core_axis_name="c", subcore_axis_name="s"),
    scratch_types=[
        pltpu.VMEM((PC, CW), jnp.int32),
        pltpu.VMEM((PC, CW), jnp.int32),
        pltpu.VMEM((CW, D), jnp.float32),
        pltpu.VMEM_SHARED((NROW, D), jnp.float32),
        pltpu.SemaphoreType.DMA,
    ],
)(_sc_segment_body)


def _dot(a, b):
    return lax.dot_general(a, b, (((1,), (0,)), ((), ())),
                           preferred_element_type=jnp.float32)


def _tc_pre_body(h_ref, parts_ref, w1_ref, b1_ref, w2_ref, b2_ref, out_ref):
    z = h_ref[...] + (parts_ref[0, 0:N, :] + parts_ref[1, 0:N, :])
    z = _dot(z, w1_ref[...]) + b1_ref[...]
    z = jnp.maximum(z, 0.0)
    z = _dot(z, w2_ref[...]) + b2_ref[...]
    out_ref[...] = jnp.maximum(z, 0.0)


def _bn_apply(h, mean, var, g_ref, beta_ref):
    return (h - mean) / jnp.sqrt(var + 1e-5) * g_ref[...] + beta_ref[...]


def _tc_bn_body(h_ref, mean_ref, var_ref, g_ref, beta_ref, out_ref):
    out_ref[...] = _bn_apply(h_ref[...], mean_ref[...], var_ref[...],
                             g_ref, beta_ref)


def _tc_final_body(h_ref, mean_ref, var_ref, g_ref, beta_ref,
                   onehot_ref, fc1w_ref, fc1b_ref, fc2w_ref, fc2b_ref,
                   out_ref):
    h = _bn_apply(h_ref[...], mean_ref[...], var_ref[...], g_ref, beta_ref)
    pooled = lax.dot_general(onehot_ref[...], h, (((0,), (0,)), ((), ())),
                             preferred_element_type=jnp.float32)
    o = _dot(pooled, fc1w_ref[...]) + fc1b_ref[...]
    o = jnp.maximum(o, 0.0)
    out_ref[...] = _dot(o, fc2w_ref[...]) + fc2b_ref[...]


def _tc_pre(h, parts, w1, b1, w2, b2):
    return pl.pallas_call(
        _tc_pre_body,
        out_shape=jax.ShapeDtypeStruct((N, D), jnp.float32),
    )(h, parts, w1, b1.reshape(1, D), w2, b2.reshape(1, D))


def _tc_bn(hp, mean, var, g, beta):
    return pl.pallas_call(
        _tc_bn_body,
        out_shape=jax.ShapeDtypeStruct((N, D), jnp.float32),
    )(hp, mean.reshape(1, D), var.reshape(1, D),
      g.reshape(1, D), beta.reshape(1, D))


def _tc_final(hp, mean, var, g, beta, onehot, fc1_w, fc1_b, fc2_w, fc2_b):
    return pl.pallas_call(
        _tc_final_body,
        out_shape=jax.ShapeDtypeStruct((NGRAPH, D), jnp.float32),
    )(hp, mean.reshape(1, D), var.reshape(1, D),
      g.reshape(1, D), beta.reshape(1, D), onehot,
      fc1_w, fc1_b.reshape(1, D), fc2_w, fc2_b.reshape(1, D))


def kernel(x, edge_index, batch, conv_w1, conv_b1, conv_w2, conv_b2,
           bn_g, bn_b, fc1_w, fc1_b, fc2_w, fc2_b):
    src = edge_index[0]
    dst = edge_index[1]
    order1 = jnp.argsort(dst, stable=True)
    ds1 = dst[order1]
    ss1 = src[order1]
    first = jnp.searchsorted(ds1, ds1, side="left").astype(jnp.int32)
    rank = jnp.arange(E, dtype=jnp.int32) - first
    wid = (jnp.arange(E, dtype=jnp.int32) // EPW)
    key = wid * jnp.int32(16384) + jnp.minimum(rank, 16383)
    order2 = jnp.argsort(key, stable=True)
    k2 = key[order2]
    ds2 = ds1[order2]
    ss2 = ss1[order2]
    gfirst = jnp.searchsorted(k2, k2, side="left").astype(jnp.int32)
    glast = jnp.searchsorted(k2, k2, side="right").astype(jnp.int32)
    gidx = jnp.arange(E, dtype=jnp.int32) - gfirst
    gcount = glast - gfirst
    psize = ((gcount + (CW - 1)) // CW) * CW
    contrib = jnp.where(jnp.arange(E, dtype=jnp.int32) == gfirst, psize, 0)
    csum = jnp.cumsum(contrib)
    before_group = csum - psize
    wbase = before_group[jnp.arange(NW) * EPW]
    pos = wid * jnp.int32(EWP) + (before_group - wbase[wid]) + gidx
    src_flat = jnp.zeros((NW * EWP,), jnp.int32).at[pos].set(
        ss2, mode="drop")
    dst_flat = jnp.full((NW * EWP,), DEAD, jnp.int32).at[pos].set(
        ds2, mode="drop")
    src_p = src_flat.reshape(NW, NCHUNK, CW)
    dst_p = dst_flat.reshape(NW, NCHUNK, CW)
    zeros = jnp.zeros((NROW, D), jnp.float32)
    onehot = (batch[:, None] == jnp.arange(NGRAPH, dtype=jnp.int32)[None, :]
              ).astype(jnp.float32)

    h = x.astype(jnp.float32)
    for i in range(5):
        parts = _sc_segment(h, zeros, src_p, dst_p)
        hp = _tc_pre(h, parts, conv_w1[i], conv_b1[i], conv_w2[i],
                     conv_b2[i])
        mean = jnp.mean(hp, axis=0)
        var = jnp.var(hp, axis=0)
        if i < 4:
            h = _tc_bn(hp, mean, var, bn_g[i], bn_b[i])
        else:
            return _tc_final(hp, mean, var, bn_g[4], bn_b[4], onehot,
                             fc1_w, fc1_b, fc2_w, fc2_b)
